# Initial kernel scaffold; baseline (speedup 1.0000x reference)
import jax
import jax.numpy as jnp
from jax.experimental import pallas as pl


def kernel(x, edge_index, params):
    raise NotImplementedError("write your pallas kernel here")


# speedup vs baseline: 13.4816x; 13.4816x over previous
import functools

import jax
import jax.numpy as jnp
from jax import lax
from jax.experimental import pallas as pl
from jax.experimental.pallas import tpu as pltpu
from jax.experimental.pallas import tpu_sc as plsc

N = 10000
E = 320000
D = 128

NTILE = 16
NCORE = 2
NWORK = NTILE * NCORE

NPAD = 10240
ROWB = 1280
NBLK = NPAD // ROWB
ROWS_PT = NPAD // NTILE

CHUNK = 128
EPT = 10112
NCHUNK = EPT // CHUNK
EPAD = NWORK * EPT

_BN_C = (1.0 + 1e-5) ** -0.5


_sc_mesh = plsc.VectorSubcoreMesh(core_axis_name="c", subcore_axis_name="s")


@functools.partial(
    pl.kernel,
    mesh=_sc_mesh,
    out_type=jax.ShapeDtypeStruct((NCORE, NPAD), jnp.float32),
    scratch_types=[
        pltpu.VMEM((CHUNK,), jnp.int32),
        pltpu.VMEM((CHUNK,), jnp.float32),
        pltpu.VMEM((ROWS_PT,), jnp.float32),
        pltpu.VMEM_SHARED((NPAD,), jnp.float32),
    ],
)
def _deg_kernel(dst_hbm, out_hbm, idx_v, ones_v, zrow_v, acc):
    c = lax.axis_index("c")
    s = lax.axis_index("s")
    w = c * NTILE + s

    for j in range(CHUNK // 16):
        ones_v[pl.ds(j * 16, 16)] = jnp.ones((16,), jnp.float32)

    def _zero(i, carry):
        zrow_v[pl.ds(i * 16, 16)] = jnp.zeros((16,), jnp.float32)
        return carry

    lax.fori_loop(0, ROWS_PT // 16, _zero, 0)
    pltpu.sync_copy(zrow_v, acc.at[pl.ds(s * ROWS_PT, ROWS_PT)])
    plsc.subcore_barrier()

    def _step(k, carry):
        eoff = pl.multiple_of(w * EPT + k * CHUNK, CHUNK)
        pltpu.sync_copy(dst_hbm.at[pl.ds(eoff, CHUNK)], idx_v)
        pltpu.sync_copy(ones_v, acc.at[idx_v], add=True)
        return carry

    lax.fori_loop(0, NCHUNK, _step, 0)
    plsc.subcore_barrier()
    pltpu.sync_copy(acc.at[pl.ds(s * ROWS_PT, ROWS_PT)],
                    out_hbm.at[c, pl.ds(s * ROWS_PT, ROWS_PT)])


@functools.partial(
    pl.kernel,
    mesh=_sc_mesh,
    out_type=jax.ShapeDtypeStruct((NCORE, NPAD, D), jnp.float32),
    scratch_types=[
        pltpu.VMEM((CHUNK,), jnp.int32),
        pltpu.VMEM((CHUNK,), jnp.int32),
        pltpu.VMEM((CHUNK, D), jnp.float32),
        pltpu.VMEM_SHARED((NPAD, D), jnp.float32),
        pltpu.SemaphoreType.DMA,
    ],
)
def _edge_kernel(g_hbm, src_hbm, dst_hbm, out_hbm, src_v, dst_v, rows_v, acc, sem):
    c = lax.axis_index("c")
    s = lax.axis_index("s")
    w = c * NTILE + s

    def _zero(t, carry):
        i = t // (D // 16)
        j = t % (D // 16)
        rows_v[i, pl.ds(j * 16, 16)] = jnp.zeros((16,), jnp.float32)
        return carry

    lax.fori_loop(0, CHUNK * (D // 16), _zero, 0)
    for j in range(ROWS_PT // CHUNK):
        pltpu.sync_copy(rows_v, acc.at[pl.ds(s * ROWS_PT + j * CHUNK, CHUNK)])
    plsc.subcore_barrier()

    def _step(k, carry):
        eoff = pl.multiple_of(w * EPT + k * CHUNK, CHUNK)
        pltpu.sync_copy(src_hbm.at[pl.ds(eoff, CHUNK)], src_v)
        pltpu.sync_copy(dst_hbm.at[pl.ds(eoff, CHUNK)], dst_v)
        pltpu.async_copy(g_hbm.at[src_v], rows_v, sem).wait()
        pltpu.sync_copy(rows_v, acc.at[dst_v], add=True)
        return carry

    lax.fori_loop(0, NCHUNK, _step, 0)
    plsc.subcore_barrier()
    pltpu.sync_copy(acc.at[pl.ds(s * ROWS_PT, ROWS_PT)],
                    out_hbm.at[c, pl.ds(s * ROWS_PT, ROWS_PT)])


def _dense0_body(x_ref, p0_ref, p1_ref, w0, b0, n0g, n0b, w1, b1, n1g, n1b,
                 wc0, g0_ref, dinv_ref):
    dinv = lax.rsqrt(p0_ref[...] + p1_ref[...] + 1.0)
    h = jnp.dot(x_ref[...], w0[...], preferred_element_type=jnp.float32) + b0[...]
    h = jnp.maximum(h * n0g[...] + n0b[...], 0.0)
    h = jnp.dot(h, w1[...], preferred_element_type=jnp.float32) + b1[...]
    h = jnp.maximum(h * n1g[...] + n1b[...], 0.0)
    g0_ref[...] = jnp.dot(h, wc0[...], preferred_element_type=jnp.float32) * dinv
    dinv_ref[...] = dinv


def _mid_body(p0, p1, g, dinv, bc, ng, nb, wn, gn_ref):
    z = (p0[...] + p1[...] + g[...]) * dinv[...] + bc[...]
    h = jnp.maximum(z * ng[...] + nb[...], 0.0)
    gn_ref[...] = jnp.dot(h, wn[...], preferred_element_type=jnp.float32) * dinv[...]


def _final_body(p0, p1, g, dinv, bc, out_ref):
    t = (p0[...] + p1[...] + g[...]) * dinv[...] + bc[...]
    m = jnp.max(t, axis=1, keepdims=True)
    lse = jnp.log(jnp.sum(jnp.exp(t - m), axis=1, keepdims=True)) + m
    out_ref[...] = t - lse


def _row_spec():
    return pl.BlockSpec((ROWB, D), lambda i: (i, 0))


def _col_spec():
    return pl.BlockSpec((ROWB, 1), lambda i: (i, 0))


def _w_spec():
    return pl.BlockSpec((D, D), lambda i: (0, 0))


def _v_spec():
    return pl.BlockSpec((1, D), lambda i: (0, 0))


def _dense0(xp, p0, p1, w0, b0, n0g, n0b, w1, b1, n1g, n1b, wc0):
    return pl.pallas_call(
        _dense0_body,
        grid=(NBLK,),
        in_specs=[_row_spec(), _col_spec(), _col_spec(),
                  _w_spec(), _v_spec(), _v_spec(), _v_spec(),
                  _w_spec(), _v_spec(), _v_spec(), _v_spec(),
                  _w_spec()],
        out_specs=[_row_spec(), _col_spec()],
        out_shape=[jax.ShapeDtypeStruct((NPAD, D), jnp.float32),
                   jax.ShapeDtypeStruct((NPAD, 1), jnp.float32)],
    )(xp, p0, p1, w0, b0, n0g, n0b, w1, b1, n1g, n1b, wc0)


def _mid(p0, p1, g, dinv, bc, ng, nb, wn):
    return pl.pallas_call(
        _mid_body,
        grid=(NBLK,),
        in_specs=[_row_spec(), _row_spec(), _row_spec(), _col_spec(),
                  _v_spec(), _v_spec(), _v_spec(), _w_spec()],
        out_specs=_row_spec(),
        out_shape=jax.ShapeDtypeStruct((NPAD, D), jnp.float32),
    )(p0, p1, g, dinv, bc, ng, nb, wn)


def _final(p0, p1, g, dinv, bc):
    return pl.pallas_call(
        _final_body,
        grid=(NBLK,),
        in_specs=[_row_spec(), _row_spec(), _row_spec(), _col_spec(),
                  _v_spec()],
        out_specs=_row_spec(),
        out_shape=jax.ShapeDtypeStruct((NPAD, D), jnp.float32),
    )(p0, p1, g, dinv, bc)


def kernel(x, edge_index, params):
    p = params
    src = edge_index[0]
    dst = edge_index[1]

    padn = EPAD - E
    ar = jnp.arange(padn, dtype=jnp.int32)
    srcp = jnp.concatenate([src, ar % N])
    dstp = jnp.concatenate([dst, N + (ar % (NPAD - N))])

    xp = jnp.zeros((NPAD, D), jnp.float32).at[:N].set(x)

    row = lambda v: v.reshape(1, D)
    b0 = row(p["lin0_b"])
    b1 = row(p["lin1_b"])
    n0g = row(p["bn0_g"] * _BN_C)
    n0b = row(p["bn0_b"])
    n1g = row(p["bn1_g"] * _BN_C)
    n1b = row(p["bn1_b"])
    n2g = row(p["bn2_g"] * _BN_C)
    n2b = row(p["bn2_b"])
    n3g = row(p["bn3_g"] * _BN_C)
    n3b = row(p["bn3_b"])
    bc0 = row(p["conv0_b"])
    bc1 = row(p["conv1_b"])
    bc2 = row(p["conv2_b"])

    degp = _deg_kernel(dstp)
    dp0 = degp[0][:, None]
    dp1 = degp[1][:, None]

    g0, dinv = _dense0(xp, dp0, dp1,
                       p["lin0_W"], b0, n0g, n0b,
                       p["lin1_W"], b1, n1g, n1b,
                       p["conv0_W"])

    s0 = _edge_kernel(g0, srcp, dstp)
    g1 = _mid(s0[0], s0[1], g0, dinv, bc0, n2g, n2b, p["conv1_W"])

    s1 = _edge_kernel(g1, srcp, dstp)
    g2 = _mid(s1[0], s1[1], g1, dinv, bc1, n3g, n3b, p["conv2_W"])

    s2 = _edge_kernel(g2, srcp, dstp)
    out = _final(s2[0], s2[1], g2, dinv, bc2)
    return out[:N]
